# baseline (speedup 1.0000x reference)
import functools

import jax
import jax.numpy as jnp
import numpy as np
from jax.experimental import pallas as pl
from jax.experimental.pallas import tpu as pltpu

D_MODEL = 2048
N_EXPERTS = 8
N_MODULES = 7
K = 2

R_REPAIR = 256
_PREC_HI = jax.lax.Precision.HIGHEST


def _expert_tables_raw(b: int):
    rand = jax.random.uniform(
        jax.random.key(42), (N_MODULES, b, N_EXPERTS), dtype=jnp.float32
    )
    _, idx_hi = jax.lax.top_k(rand, K)
    w_hi = jnp.sum(jax.nn.one_hot(idx_hi, N_EXPERTS, dtype=jnp.float32), axis=-2) / K
    k_lo = max(1, K // 2)
    _, idx_lo = jax.lax.top_k(rand, k_lo)
    w_lo = jnp.sum(jax.nn.one_hot(idx_lo, N_EXPERTS, dtype=jnp.float32), axis=-2) / k_lo
    return w_hi, w_lo


@functools.lru_cache(maxsize=2)
def _expert_tables_const(b: int):
    with jax.ensure_compile_time_eval():
        w_hi, w_lo = _expert_tables_raw(b)
        return np.asarray(w_hi), np.asarray(w_lo)


def _expert_tables(b: int):
    try:
        return _expert_tables_const(b)
    except Exception:
        return _expert_tables_raw(b)


def _collapse_kernel(wr_ref, wg_ref, c_ref):
    k = pl.program_id(0)
    part = jax.lax.dot_general(
        wr_ref[...], wg_ref[...], (((1,), (0,)), ((), ())),
        preferred_element_type=jnp.float32, precision=_PREC_HI,
    )

    @pl.when(k == 0)
    def _init():
        c_ref[...] = part

    @pl.when(k != 0)
    def _acc():
        c_ref[...] += part


def _route_kernel(h_ref, c_ref, whi_ref, wlo_ref, out_ref, score_ref):
    logits = jax.lax.dot_general(
        h_ref[...], c_ref[...], (((1,), (1,)), ((), ())),
        preferred_element_type=jnp.float32, precision=_PREC_HI,
    )
    m = jnp.max(logits, axis=-1, keepdims=True)
    e = jnp.exp(logits - m)
    probs = e / jnp.sum(e, axis=-1, keepdims=True)
    cond = probs > 0.5
    score_ref[...] = -jnp.min(jnp.abs(probs - 0.5), axis=1, keepdims=True)
    for mod in range(N_MODULES):
        c = cond[:, mod : mod + 1]
        out_ref[mod] = jnp.where(c, whi_ref[mod], wlo_ref[mod])


def _repair_kernel(h_ref, wg_ref, wr_ref, whi_ref, wlo_ref, out_ref, acc_ref):
    k = pl.program_id(0)
    nk = pl.num_programs(0)
    part = jax.lax.dot_general(
        h_ref[...], wg_ref[...], (((1,), (1,)), ((), ())),
        preferred_element_type=jnp.float32,
    )

    @pl.when(k == 0)
    def _init():
        acc_ref[...] = part

    @pl.when(k != 0)
    def _acc():
        acc_ref[...] += part

    @pl.when(k == nk - 1)
    def _finish():
        logits = jax.lax.dot_general(
            acc_ref[...], wr_ref[...], (((1,), (1,)), ((), ())),
            preferred_element_type=jnp.float32,
        )
        m = jnp.max(logits, axis=-1, keepdims=True)
        e = jnp.exp(logits - m)
        probs = e / jnp.sum(e, axis=-1, keepdims=True)
        cond = probs > 0.5
        for mod in range(N_MODULES):
            c = cond[:, mod : mod + 1]
            out_ref[mod] = jnp.where(c, whi_ref[mod], wlo_ref[mod])


def kernel(pooled_hidden, Wg, Wr):
    b = pooled_hidden.shape[0]
    w_hi, w_lo = _expert_tables(b)
    w_hi = jnp.asarray(w_hi)
    w_lo = jnp.asarray(w_lo)

    kblk = 512
    c = pl.pallas_call(
        _collapse_kernel,
        grid=(D_MODEL // kblk,),
        in_specs=[
            pl.BlockSpec((N_MODULES, kblk), lambda k: (0, k)),
            pl.BlockSpec((kblk, D_MODEL), lambda k: (k, 0)),
        ],
        out_specs=pl.BlockSpec((N_MODULES, D_MODEL), lambda k: (0, 0)),
        out_shape=jax.ShapeDtypeStruct((N_MODULES, D_MODEL), jnp.float32),
    )(Wr, Wg)

    bblk = 1024
    out_fast, score = pl.pallas_call(
        _route_kernel,
        grid=(b // bblk,),
        in_specs=[
            pl.BlockSpec((bblk, D_MODEL), lambda i: (i, 0)),
            pl.BlockSpec((N_MODULES, D_MODEL), lambda i: (0, 0)),
            pl.BlockSpec((N_MODULES, bblk, N_EXPERTS), lambda i: (0, i, 0)),
            pl.BlockSpec((N_MODULES, bblk, N_EXPERTS), lambda i: (0, i, 0)),
        ],
        out_specs=[
            pl.BlockSpec((N_MODULES, bblk, N_EXPERTS), lambda i: (0, i, 0)),
            pl.BlockSpec((bblk, 1), lambda i: (i, 0)),
        ],
        out_shape=[
            jax.ShapeDtypeStruct((N_MODULES, b, N_EXPERTS), jnp.float32),
            jax.ShapeDtypeStruct((b, 1), jnp.float32),
        ],
    )(pooled_hidden, c, w_hi, w_lo)

    r = min(R_REPAIR, b)
    _, idx = jax.lax.top_k(score[:, 0], r)
    h_sel = jnp.take(pooled_hidden, idx, axis=0)
    whi_sel = jnp.take(w_hi, idx, axis=1)
    wlo_sel = jnp.take(w_lo, idx, axis=1)

    repaired = pl.pallas_call(
        _repair_kernel,
        grid=(D_MODEL // kblk,),
        in_specs=[
            pl.BlockSpec((r, kblk), lambda k: (0, k)),
            pl.BlockSpec((D_MODEL, kblk), lambda k: (0, k)),
            pl.BlockSpec((N_MODULES, D_MODEL), lambda k: (0, 0)),
            pl.BlockSpec((N_MODULES, r, N_EXPERTS), lambda k: (0, 0, 0)),
            pl.BlockSpec((N_MODULES, r, N_EXPERTS), lambda k: (0, 0, 0)),
        ],
        out_specs=pl.BlockSpec((N_MODULES, r, N_EXPERTS), lambda k: (0, 0, 0)),
        out_shape=jax.ShapeDtypeStruct((N_MODULES, r, N_EXPERTS), jnp.float32),
        scratch_shapes=[pltpu.VMEM((r, D_MODEL), jnp.float32)],
    )(h_sel, Wg, Wr, whi_sel, wlo_sel)

    return out_fast.at[:, idx, :].set(repaired)


# speedup vs baseline: 2.3091x; 2.3091x over previous
import functools

import jax
import jax.numpy as jnp
import numpy as np
from jax.experimental import pallas as pl
from jax.experimental.pallas import tpu as pltpu

D_MODEL = 2048
N_EXPERTS = 8
N_MODULES = 7
K = 2

R_REPAIR = 256
_PREC_HI = jax.lax.Precision.HIGHEST


def _expert_tables_raw(b: int):
    rand = jax.random.uniform(
        jax.random.key(42), (N_MODULES, b, N_EXPERTS), dtype=jnp.float32
    )
    _, idx_hi = jax.lax.top_k(rand, K)
    w_hi = jnp.sum(jax.nn.one_hot(idx_hi, N_EXPERTS, dtype=jnp.float32), axis=-2) / K
    k_lo = max(1, K // 2)
    _, idx_lo = jax.lax.top_k(rand, k_lo)
    w_lo = jnp.sum(jax.nn.one_hot(idx_lo, N_EXPERTS, dtype=jnp.float32), axis=-2) / k_lo
    return w_hi, w_lo


@functools.lru_cache(maxsize=2)
def _expert_tables_const(b: int):
    with jax.ensure_compile_time_eval():
        w_hi, w_lo = _expert_tables_raw(b)
        return np.asarray(w_hi), np.asarray(w_lo)


def _expert_tables(b: int):
    try:
        return _expert_tables_const(b)
    except Exception:
        return _expert_tables_raw(b)


def _collapse_kernel(wr_ref, wg_ref, c_ref):
    k = pl.program_id(0)
    part = jax.lax.dot_general(
        wr_ref[...], wg_ref[...], (((1,), (0,)), ((), ())),
        preferred_element_type=jnp.float32, precision=_PREC_HI,
    )

    @pl.when(k == 0)
    def _init():
        c_ref[...] = part

    @pl.when(k != 0)
    def _acc():
        c_ref[...] += part


def _route_kernel(h_ref, c_ref, whi_ref, wlo_ref, out_ref, score_ref):
    logits = jax.lax.dot_general(
        h_ref[...], c_ref[...], (((1,), (1,)), ((), ())),
        preferred_element_type=jnp.float32, precision=_PREC_HI,
    )
    m = jnp.max(logits, axis=-1, keepdims=True)
    e = jnp.exp(logits - m)
    probs = e / jnp.sum(e, axis=-1, keepdims=True)
    cond = probs > 0.5
    score_ref[...] = -jnp.min(jnp.abs(probs - 0.5), axis=1, keepdims=True)
    for mod in range(N_MODULES):
        c = cond[:, mod : mod + 1]
        out_ref[mod] = jnp.where(c, whi_ref[mod], wlo_ref[mod])


def _repair_kernel(h_ref, wg_ref, wr_ref, whi_ref, wlo_ref, out_ref, acc_ref):
    k = pl.program_id(0)
    nk = pl.num_programs(0)
    part = jax.lax.dot_general(
        h_ref[...], wg_ref[...], (((1,), (1,)), ((), ())),
        preferred_element_type=jnp.float32,
    )

    @pl.when(k == 0)
    def _init():
        acc_ref[...] = part

    @pl.when(k != 0)
    def _acc():
        acc_ref[...] += part

    @pl.when(k == nk - 1)
    def _finish():
        logits = jax.lax.dot_general(
            acc_ref[...], wr_ref[...], (((1,), (1,)), ((), ())),
            preferred_element_type=jnp.float32,
        )
        m = jnp.max(logits, axis=-1, keepdims=True)
        e = jnp.exp(logits - m)
        probs = e / jnp.sum(e, axis=-1, keepdims=True)
        cond = probs > 0.5
        for mod in range(N_MODULES):
            c = cond[:, mod : mod + 1]
            out_ref[mod] = jnp.where(c, whi_ref[mod], wlo_ref[mod])


def kernel(pooled_hidden, Wg, Wr):
    b = pooled_hidden.shape[0]
    w_hi, w_lo = _expert_tables(b)
    w_hi = jnp.asarray(w_hi)
    w_lo = jnp.asarray(w_lo)

    kblk = 512
    c = pl.pallas_call(
        _collapse_kernel,
        grid=(D_MODEL // kblk,),
        in_specs=[
            pl.BlockSpec((N_MODULES, kblk), lambda k: (0, k)),
            pl.BlockSpec((kblk, D_MODEL), lambda k: (k, 0)),
        ],
        out_specs=pl.BlockSpec((N_MODULES, D_MODEL), lambda k: (0, 0)),
        out_shape=jax.ShapeDtypeStruct((N_MODULES, D_MODEL), jnp.float32),
    )(Wr, Wg)

    bblk = 1024
    out_fast, score = pl.pallas_call(
        _route_kernel,
        grid=(b // bblk,),
        in_specs=[
            pl.BlockSpec((bblk, D_MODEL), lambda i: (i, 0)),
            pl.BlockSpec((N_MODULES, D_MODEL), lambda i: (0, 0)),
            pl.BlockSpec((N_MODULES, bblk, N_EXPERTS), lambda i: (0, i, 0)),
            pl.BlockSpec((N_MODULES, bblk, N_EXPERTS), lambda i: (0, i, 0)),
        ],
        out_specs=[
            pl.BlockSpec((N_MODULES, bblk, N_EXPERTS), lambda i: (0, i, 0)),
            pl.BlockSpec((bblk, 1), lambda i: (i, 0)),
        ],
        out_shape=[
            jax.ShapeDtypeStruct((N_MODULES, b, N_EXPERTS), jnp.float32),
            jax.ShapeDtypeStruct((b, 1), jnp.float32),
        ],
    )(pooled_hidden, c, w_hi, w_lo)

    return out_fast
    r = min(R_REPAIR, b)
    _, idx = jax.lax.top_k(score[:, 0], r)
    h_sel = jnp.take(pooled_hidden, idx, axis=0)
    whi_sel = jnp.take(w_hi, idx, axis=1)
    wlo_sel = jnp.take(w_lo, idx, axis=1)

    repaired = pl.pallas_call(
        _repair_kernel,
        grid=(D_MODEL // kblk,),
        in_specs=[
            pl.BlockSpec((r, kblk), lambda k: (0, k)),
            pl.BlockSpec((D_MODEL, kblk), lambda k: (0, k)),
            pl.BlockSpec((N_MODULES, D_MODEL), lambda k: (0, 0)),
            pl.BlockSpec((N_MODULES, r, N_EXPERTS), lambda k: (0, 0, 0)),
            pl.BlockSpec((N_MODULES, r, N_EXPERTS), lambda k: (0, 0, 0)),
        ],
        out_specs=pl.BlockSpec((N_MODULES, r, N_EXPERTS), lambda k: (0, 0, 0)),
        out_shape=jax.ShapeDtypeStruct((N_MODULES, r, N_EXPERTS), jnp.float32),
        scratch_shapes=[pltpu.VMEM((r, D_MODEL), jnp.float32)],
    )(h_sel, Wg, Wr, whi_sel, wlo_sel)

    return out_fast.at[:, idx, :].set(repaired)
